# Initial kernel scaffold; baseline (speedup 1.0000x reference)
import functools

import jax
import jax.numpy as jnp
from jax import lax
from jax.experimental import pallas as pl
from jax.experimental.pallas import tpu as pltpu
from jax.experimental.pallas import tpu_sc as plsc

D = 64
NSUB = 16
NCORE = 2
G = 1536
SUB = G // 128


def _bcast16(v, i):
    return lax.gather(
        v,
        jnp.full((16, 1), i, jnp.int32),
        lax.GatherDimensionNumbers(
            offset_dims=(), collapsed_slice_dims=(0,), start_index_map=(0,)),
        (1,),
        mode=lax.GatherScatterMode.PROMISE_IN_BOUNDS)


def _make_prop(half_real, half_pad, ept, d):
    nb = ept // G
    rows_pt = half_pad // NSUB
    npad = 2 * half_pad
    mesh = plsc.VectorSubcoreMesh(core_axis_name="c", subcore_axis_name="s")

    @functools.partial(
        pl.kernel,
        mesh=mesh,
        out_type=jax.ShapeDtypeStruct((npad, d), jnp.float32),
        scratch_types=[
            pltpu.VMEM((SUB, 128), jnp.int32),
            pltpu.VMEM((SUB, 128), jnp.int32),
            pltpu.VMEM((SUB, 128), jnp.float32),
            pltpu.VMEM((SUB, 128), jnp.int32),
            pltpu.VMEM((G, d), jnp.float32),
            pltpu.VMEM_SHARED((half_pad, d), jnp.float32),
            pltpu.SemaphoreType.DMA,
        ],
    )
    def prop(t_hbm, src_hbm, dst_hbm, val_hbm, z_hbm, out_hbm,
             src_v, dst_v, val_v, loc_v, rows_v, acc_sh, sem):
        c = lax.axis_index("c")
        s = lax.axis_index("s")

        pltpu.sync_copy(z_hbm, acc_sh.at[pl.ds(s * rows_pt, rows_pt)])
        plsc.subcore_barrier()

        base_node = c * half_real
        dummy_row = half_real + s
        t_rowbase = s * (ept // 128)

        def batch(b, carry):
            row_off = t_rowbase + b * SUB
            pltpu.sync_copy(src_hbm.at[pl.ds(row_off, SUB)], src_v)
            pltpu.sync_copy(dst_hbm.at[pl.ds(row_off, SUB)], dst_v)
            pltpu.sync_copy(val_hbm.at[pl.ds(row_off, SUB)], val_v)
            handles = []
            for sb in range(SUB):
                handles.append(pltpu.async_copy(
                    t_hbm.at[src_v.at[sb]],
                    rows_v.at[pl.ds(sb * 128, 128)], sem))
            for h in handles:
                h.wait()

            def group(r, _):
                for cc in range(8):
                    col = cc * 16
                    dst16 = dst_v[r, pl.ds(col, 16)]
                    val16 = val_v[r, pl.ds(col, 16)]
                    loc = dst16 - base_node
                    inb = (loc >= 0) & (loc < half_real)
                    loc_v[r, pl.ds(col, 16)] = jnp.where(inb, loc, dummy_row)
                    for i in range(16):
                        bv = _bcast16(val16, i)
                        e = r * 128 + col + i
                        for jj in range(d // 16):
                            rows_v[e, pl.ds(jj * 16, 16)] = (
                                rows_v[e, pl.ds(jj * 16, 16)] * bv)
                return 0

            lax.fori_loop(0, SUB, group, 0)

            for sb in range(SUB):
                pltpu.sync_copy(
                    rows_v.at[pl.ds(sb * 128, 128)],
                    acc_sh.at[loc_v.at[sb]], add=True)
            return 0

        lax.fori_loop(0, nb, batch, 0)

        plsc.subcore_barrier()
        pltpu.sync_copy(
            acc_sh.at[pl.ds(s * rows_pt, rows_pt)],
            out_hbm.at[pl.ds(c * half_pad + s * rows_pt, rows_pt)])

    return prop


def _make_gather_mean(npad, nidx, d):
    per_tile = nidx // (NCORE * NSUB)
    idx_rows = per_tile // 128
    mesh = plsc.VectorSubcoreMesh(core_axis_name="c", subcore_axis_name="s")

    @functools.partial(
        pl.kernel,
        mesh=mesh,
        out_type=jax.ShapeDtypeStruct((nidx, d), jnp.float32),
        scratch_types=[
            pltpu.VMEM((idx_rows, 128), jnp.int32),
            pltpu.VMEM((4 * per_tile, d), jnp.float32),
            pltpu.VMEM((per_tile, d), jnp.float32),
            pltpu.SemaphoreType.DMA,
        ],
    )
    def gmean(t0, t1, t2, t3, gidx_hbm, out_hbm, idx_v, tbl_v, out_v, sem):
        c = lax.axis_index("c")
        s = lax.axis_index("s")
        wid = c * NSUB + s
        pltpu.sync_copy(gidx_hbm.at[pl.ds(wid * idx_rows, idx_rows)], idx_v)
        handles = []
        for k, t in enumerate((t0, t1, t2, t3)):
            for sb in range(idx_rows):
                handles.append(pltpu.async_copy(
                    t.at[idx_v.at[sb]],
                    tbl_v.at[pl.ds(k * per_tile + sb * 128, 128)], sem))
        for h in handles:
            h.wait()

        def row(r, _):
            for jj in range(d // 16):
                sl = pl.ds(jj * 16, 16)
                acc = (tbl_v[r, sl] + tbl_v[per_tile + r, sl]
                       + tbl_v[2 * per_tile + r, sl]
                       + tbl_v[3 * per_tile + r, sl])
                out_v[r, sl] = acc * 0.25
            return 0

        lax.fori_loop(0, per_tile, row, 0)
        pltpu.sync_copy(out_v, out_hbm.at[pl.ds(wid * per_tile, per_tile)])

    return gmean


def _dense_body(ue_ref, ie_ref, wk_ref, bk_ref, ws_ref, out_ref):
    ue = ue_ref[...]
    ie = ie_ref[...]
    nf = wk_ref.shape[0]

    def factors(x):
        fs = []
        for k in range(nf):
            w = wk_ref[k]
            f = lax.dot_general(
                x, w, (((1,), (1,)), ((), ())),
                precision=lax.Precision.HIGHEST,
                preferred_element_type=jnp.float32)
            f = jnp.maximum(f + bk_ref[k][None, :], 0.0)
            n = jnp.sqrt(jnp.sum(f * f, axis=1, keepdims=True))
            fs.append(f / jnp.maximum(n, 1e-12))
        return fs

    uf = factors(ue)
    itf = factors(ie)
    acc = jnp.zeros((ue.shape[0],), jnp.float32)
    for i in range(nf):
        for j in range(nf):
            acc = acc + ws_ref[i, j] * jnp.sum(uf[i] * itf[j], axis=1)
    out_ref[...] = acc


def _dense_scores(ue, ie, wk, bk, ws):
    b, d = ue.shape
    bs = 512
    nf = wk.shape[0]
    return pl.pallas_call(
        _dense_body,
        grid=(b // bs,),
        in_specs=[
            pl.BlockSpec((bs, d), lambda i: (i, 0)),
            pl.BlockSpec((bs, d), lambda i: (i, 0)),
            pl.BlockSpec((nf, d, d), lambda i: (0, 0, 0)),
            pl.BlockSpec((nf, d), lambda i: (0, 0)),
            pl.BlockSpec((nf, nf), lambda i: (0, 0)),
        ],
        out_specs=pl.BlockSpec((bs,), lambda i: (i,)),
        out_shape=jax.ShapeDtypeStruct((b,), jnp.float32),
    )(ue, ie, wk, bk, ws)


def kernel(users, items, user_emb, item_emb, edge_index, edge_vals, Wk, bk, W_s):
    nu, d = user_emb.shape
    ni = item_emb.shape[0]
    e = edge_index.shape[1]
    bsz = users.shape[0]
    assert nu == ni
    half_real = nu
    half_pad = ((nu + NSUB + 127) // 128) * 128
    npad = 2 * half_pad
    gap = half_pad - half_real

    zpad = jnp.zeros((gap, d), jnp.float32)
    t0 = jnp.concatenate([user_emb, zpad, item_emb, zpad], axis=0)

    src = edge_index[0]
    dst = edge_index[1]
    src_p = src + jnp.where(src >= half_real, gap, 0).astype(jnp.int32)

    ept = ((e // NSUB + G - 1) // G) * G
    e_pad = ept * NSUB
    pad_n = e_pad - e
    src_p = jnp.pad(src_p, (0, pad_n)).reshape(e_pad // 128, 128)
    dst_p = jnp.pad(dst, (0, pad_n)).reshape(e_pad // 128, 128)
    val_p = jnp.pad(edge_vals, (0, pad_n)).reshape(e_pad // 128, 128)

    zrows = jnp.zeros((half_pad // NSUB, d), jnp.float32)

    prop = _make_prop(half_real, half_pad, ept, d)
    t1 = prop(t0, src_p, dst_p, val_p, zrows)
    t2 = prop(t1, src_p, dst_p, val_p, zrows)
    t3 = prop(t2, src_p, dst_p, val_p, zrows)

    gidx = jnp.concatenate([users, items + half_pad]).reshape(-1, 128)
    gmean = _make_gather_mean(npad, 2 * bsz, d)
    ui = gmean(t0, t1, t2, t3, gidx)

    return _dense_scores(ui[:bsz], ui[bsz:], Wk, bk, W_s)


# speedup vs baseline: 4.3534x; 4.3534x over previous
import functools

import jax
import jax.numpy as jnp
from jax import lax
from jax.experimental import pallas as pl
from jax.experimental.pallas import tpu as pltpu
from jax.experimental.pallas import tpu_sc as plsc

D = 64
NSUB = 16
NCORE = 2
G = 384
SUB = G // 128


def _bcast16(v, i):
    return lax.gather(
        v,
        jnp.full((16, 1), i, jnp.int32),
        lax.GatherDimensionNumbers(
            offset_dims=(), collapsed_slice_dims=(0,), start_index_map=(0,)),
        (1,),
        mode=lax.GatherScatterMode.PROMISE_IN_BOUNDS)


def _make_prop(half_real, half_pad, ept, d):
    nb = ept // G
    rows_pt = half_pad // NSUB
    npad = 2 * half_pad
    mesh = plsc.VectorSubcoreMesh(core_axis_name="c", subcore_axis_name="s")

    @functools.partial(
        pl.kernel,
        mesh=mesh,
        out_type=jax.ShapeDtypeStruct((npad, d), jnp.float32),
        compiler_params=pltpu.CompilerParams(use_tc_tiling_on_sc=False),
        scratch_types=[
            pltpu.VMEM((G,), jnp.int32),
            pltpu.VMEM((G,), jnp.int32),
            pltpu.VMEM((G,), jnp.float32),
            pltpu.VMEM((SUB, 128), jnp.int32),
            pltpu.VMEM((G, d), jnp.float32),
            pltpu.VMEM_SHARED((half_pad, d), jnp.float32),
            pltpu.SemaphoreType.DMA,
        ],
    )
    def prop(t_hbm, src_hbm, dst_hbm, val_hbm, z_hbm, out_hbm,
             src_v, dst_v, val_v, loc_v, rows_v, acc_sh, sem):
        c = lax.axis_index("c")
        s = lax.axis_index("s")

        pltpu.sync_copy(z_hbm, acc_sh.at[pl.ds(s * rows_pt, rows_pt)])
        plsc.subcore_barrier()

        base_node = c * half_real
        dummy_row = half_real + s
        t_base = s * ept

        def batch(b, carry):
            eoff = t_base + b * G
            pltpu.sync_copy(src_hbm.at[pl.ds(eoff, G)], src_v)
            pltpu.sync_copy(dst_hbm.at[pl.ds(eoff, G)], dst_v)
            pltpu.sync_copy(val_hbm.at[pl.ds(eoff, G)], val_v)
            handles = []
            for sb in range(SUB):
                handles.append(pltpu.async_copy(
                    t_hbm.at[src_v.at[pl.ds(sb * 128, 128)]],
                    rows_v.at[pl.ds(sb * 128, 128)], sem))
            for h in handles:
                h.wait()

            def group(r, _):
                for cc in range(8):
                    col = r * 128 + cc * 16
                    dst16 = dst_v[pl.ds(col, 16)]
                    val16 = val_v[pl.ds(col, 16)]
                    loc = dst16 - base_node
                    inb = (loc >= 0) & (loc < half_real)
                    loc_v[r, pl.ds(cc * 16, 16)] = jnp.where(
                        inb, loc, dummy_row)
                    for i in range(16):
                        bv = _bcast16(val16, i)
                        e = col + i
                        for jj in range(d // 16):
                            rows_v[e, pl.ds(jj * 16, 16)] = (
                                rows_v[e, pl.ds(jj * 16, 16)] * bv)
                return 0

            lax.fori_loop(0, SUB, group, 0)

            for sb in range(SUB):
                pltpu.sync_copy(
                    rows_v.at[pl.ds(sb * 128, 128)],
                    acc_sh.at[loc_v.at[sb]], add=True)
            return 0

        lax.fori_loop(0, nb, batch, 0)

        plsc.subcore_barrier()
        pltpu.sync_copy(
            acc_sh.at[pl.ds(s * rows_pt, rows_pt)],
            out_hbm.at[pl.ds(c * half_pad + s * rows_pt, rows_pt)])

    return prop


def _make_gather_mean(npad, nidx, d):
    per_tile = nidx // (NCORE * NSUB)
    idx_rows = per_tile // 128
    mesh = plsc.VectorSubcoreMesh(core_axis_name="c", subcore_axis_name="s")

    @functools.partial(
        pl.kernel,
        mesh=mesh,
        out_type=jax.ShapeDtypeStruct((nidx, d), jnp.float32),
        compiler_params=pltpu.CompilerParams(use_tc_tiling_on_sc=False),
        scratch_types=[
            pltpu.VMEM((per_tile,), jnp.int32),
            pltpu.VMEM((4 * per_tile, d), jnp.float32),
            pltpu.VMEM((per_tile, d), jnp.float32),
            pltpu.SemaphoreType.DMA,
        ],
    )
    def gmean(t0, t1, t2, t3, gidx_hbm, out_hbm, idx_v, tbl_v, out_v, sem):
        c = lax.axis_index("c")
        s = lax.axis_index("s")
        wid = c * NSUB + s
        pltpu.sync_copy(gidx_hbm.at[pl.ds(wid * per_tile, per_tile)], idx_v)
        handles = []
        for k, t in enumerate((t0, t1, t2, t3)):
            for sb in range(idx_rows):
                handles.append(pltpu.async_copy(
                    t.at[idx_v.at[pl.ds(sb * 128, 128)]],
                    tbl_v.at[pl.ds(k * per_tile + sb * 128, 128)], sem))
        for h in handles:
            h.wait()

        def row(r, _):
            for jj in range(d // 16):
                sl = pl.ds(jj * 16, 16)
                acc = (tbl_v[r, sl] + tbl_v[per_tile + r, sl]
                       + tbl_v[2 * per_tile + r, sl]
                       + tbl_v[3 * per_tile + r, sl])
                out_v[r, sl] = acc * 0.25
            return 0

        lax.fori_loop(0, per_tile, row, 0)
        pltpu.sync_copy(out_v, out_hbm.at[pl.ds(wid * per_tile, per_tile)])

    return gmean


def _dense_body(ue_ref, ie_ref, wk_ref, bk_ref, ws_ref, out_ref):
    ue = ue_ref[...]
    ie = ie_ref[...]
    nf = wk_ref.shape[0]

    def factors(x):
        fs = []
        for k in range(nf):
            w = wk_ref[k]
            f = lax.dot_general(
                x, w, (((1,), (1,)), ((), ())),
                precision=lax.Precision.HIGHEST,
                preferred_element_type=jnp.float32)
            f = jnp.maximum(f + bk_ref[k][None, :], 0.0)
            n = jnp.sqrt(jnp.sum(f * f, axis=1, keepdims=True))
            fs.append(f / jnp.maximum(n, 1e-12))
        return fs

    uf = factors(ue)
    itf = factors(ie)
    acc = jnp.zeros((ue.shape[0],), jnp.float32)
    for i in range(nf):
        for j in range(nf):
            acc = acc + ws_ref[i, j] * jnp.sum(uf[i] * itf[j], axis=1)
    out_ref[...] = acc


def _dense_scores(ue, ie, wk, bk, ws):
    b, d = ue.shape
    bs = 512
    nf = wk.shape[0]
    return pl.pallas_call(
        _dense_body,
        grid=(b // bs,),
        in_specs=[
            pl.BlockSpec((bs, d), lambda i: (i, 0)),
            pl.BlockSpec((bs, d), lambda i: (i, 0)),
            pl.BlockSpec((nf, d, d), lambda i: (0, 0, 0)),
            pl.BlockSpec((nf, d), lambda i: (0, 0)),
            pl.BlockSpec((nf, nf), lambda i: (0, 0)),
        ],
        out_specs=pl.BlockSpec((bs,), lambda i: (i,)),
        out_shape=jax.ShapeDtypeStruct((b,), jnp.float32),
    )(ue, ie, wk, bk, ws)


def kernel(users, items, user_emb, item_emb, edge_index, edge_vals, Wk, bk, W_s):
    nu, d = user_emb.shape
    ni = item_emb.shape[0]
    e = edge_index.shape[1]
    bsz = users.shape[0]
    assert nu == ni
    half_real = nu
    half_pad = ((nu + NSUB + 127) // 128) * 128
    npad = 2 * half_pad
    gap = half_pad - half_real

    zpad = jnp.zeros((gap, d), jnp.float32)
    t0 = jnp.concatenate([user_emb, zpad, item_emb, zpad], axis=0)

    src = edge_index[0]
    dst = edge_index[1]
    src_p = src + jnp.where(src >= half_real, gap, 0).astype(jnp.int32)

    ept = ((e // NSUB + G - 1) // G) * G
    e_pad = ept * NSUB
    pad_n = e_pad - e
    src_p = jnp.pad(src_p, (0, pad_n))
    dst_p = jnp.pad(dst, (0, pad_n))
    val_p = jnp.pad(edge_vals, (0, pad_n))

    zrows = jnp.zeros((half_pad // NSUB, d), jnp.float32)

    prop = _make_prop(half_real, half_pad, ept, d)
    t1 = prop(t0, src_p, dst_p, val_p, zrows)
    t2 = prop(t1, src_p, dst_p, val_p, zrows)
    t3 = prop(t2, src_p, dst_p, val_p, zrows)

    gidx = jnp.concatenate([users, items + half_pad])
    gmean = _make_gather_mean(npad, 2 * bsz, d)
    ui = gmean(t0, t1, t2, t3, gidx)

    return _dense_scores(ui[:bsz], ui[bsz:], Wk, bk, W_s)
